# baseline (speedup 1.0000x reference)
import jax
import jax.numpy as jnp
from jax import lax
from jax.experimental import pallas as pl
from jax.experimental.pallas import tpu as pltpu
from jax.experimental.pallas import tpu_sc as plsc

B = 16384
L = 200
NUM_W = 1000001
NUM_REL = 1000

NC, NS, LANES = 2, 16, 16
CHUNK = 32
CW = CHUNK * L
NGROUP = CHUNK // LANES

H_TILES_PER_SC = 2
H_ROWS = 288
S_ROWS = 544
NH_CHUNK = H_ROWS // CHUNK
NS_CHUNK = S_ROWS // CHUNK
S_BASE = 2 * H_TILES_PER_SC * H_ROWS

W_SLICE = 62504
NUM_W_PAD = W_SLICE * NS
SCW = 6400
N_STAGE = -(-W_SLICE // SCW)
STAGE_TAIL = W_SLICE - (N_STAGE - 1) * SCW


def _body(rules_hbm, rel_hbm, w_hbm, bias_hbm, out_hbm,
          rules_a, rules_b, vals_a, vals_b,
          bounce_a, bounce_b, bias_v, rel_v, out_acc,
          w_spmem, rsem_a, rsem_b, gsem_a, gsem_b, hsem, ssem):
  sid = lax.axis_index("s")
  core = lax.axis_index("c")
  is_h = sid < H_TILES_PER_SC
  ih = core * H_TILES_PER_SC + sid
  is_ = core * (NS - H_TILES_PER_SC) + (sid - H_TILES_PER_SC)
  base = jnp.where(is_h, ih * H_ROWS, S_BASE + is_ * S_ROWS)

  row_stride = lax.iota(jnp.int32, LANES) * L
  base_idx = [row_stride + g * (LANES * L) for g in range(NGROUP)]
  zero = jnp.zeros((LANES,), jnp.float32)

  rules_bufs = [rules_a, rules_b]
  vals_bufs = [vals_a, vals_b]
  rsem = [rsem_a, rsem_b]
  gsem = [gsem_a, gsem_b]
  bounce = [bounce_a, bounce_b]
  stage_n = [SCW] * (N_STAGE - 1) + [STAGE_TAIL]

  h_h = {}

  def rules_desc(c):
    p = c % 2
    return pltpu.make_async_copy(
        rules_hbm.at[pl.ds(pl.multiple_of((base + c * CHUNK) * L, 8), CW)],
        rules_bufs[p], rsem[p])

  def gather_desc(c, src):
    p = c % 2
    return pltpu.make_async_copy(src.at[rules_bufs[p]], vals_bufs[p],
                                 gsem[p])

  def issue_stage_read(k):
    h_h[k] = pltpu.async_copy(
        w_hbm.at[pl.ds(sid * W_SLICE + k * SCW, stage_n[k])],
        bounce[k % 2].at[pl.ds(0, stage_n[k])], hsem)

  def compute_chunk(c, vals_ref):
    def l_body(i, accs, vals_ref=vals_ref):
      a00, a01, a10, a11 = accs
      a00 = a00 + plsc.load_gather(vals_ref, [base_idx[0] + 2 * i])
      a01 = a01 + plsc.load_gather(vals_ref, [base_idx[0] + 2 * i + 1])
      a10 = a10 + plsc.load_gather(vals_ref, [base_idx[1] + 2 * i])
      a11 = a11 + plsc.load_gather(vals_ref, [base_idx[1] + 2 * i + 1])
      return a00, a01, a10, a11

    a00, a01, a10, a11 = lax.fori_loop(
        0, L // 2, l_body, (zero,) * 4, unroll=4)
    out_acc[pl.ds(c * CHUNK, LANES)] = a00 + a01
    out_acc[pl.ds(c * CHUNK + LANES, LANES)] = a10 + a11

  def run_worker(nchunk, src, gathers_issued):
    nrows = nchunk * CHUNK
    pltpu.sync_copy(rel_hbm.at[pl.ds(pl.multiple_of(base, 8), nrows)],
                    rel_v.at[pl.ds(0, nrows)])
    for c in range(nchunk):
      p = c % 2
      if gathers_issued <= c + 1 < nchunk:
        rules_desc(c + 1).wait()
        gather_desc(c + 1, src).start()
      gather_desc(c, src).wait()
      if c + 2 < nchunk:
        rules_desc(c + 2).start()
      compute_chunk(c, vals_bufs[p])

    for g in range(nrows // LANES):
      idx = rel_v[pl.ds(g * LANES, LANES)]
      out_acc[pl.ds(g * LANES, LANES)] = (
          out_acc[pl.ds(g * LANES, LANES)] + plsc.load_gather(bias_v, [idx]))

    pltpu.sync_copy(
        out_acc.at[pl.ds(0, nrows)],
        out_hbm.at[pl.ds(pl.multiple_of(base, 8), nrows)])

  rules_desc(0).start()
  rules_desc(1).start()
  issue_stage_read(0)
  issue_stage_read(1)
  pltpu.sync_copy(bias_hbm, bias_v)

  @pl.when(is_h)
  def _():
    rules_desc(0).wait()
    gather_desc(0, w_hbm).start()
    rules_desc(1).wait()
    gather_desc(1, w_hbm).start()

  for k in range(N_STAGE):
    h_h[k].wait()
    s = pltpu.async_copy(
        bounce[k % 2].at[pl.ds(0, stage_n[k])],
        w_spmem.at[pl.ds(sid * W_SLICE + k * SCW, stage_n[k])], ssem)
    s.wait()
    if k + 2 < N_STAGE:
      issue_stage_read(k + 2)

  plsc.subcore_barrier()

  @pl.when(is_h)
  def _():
    run_worker(NH_CHUNK, w_hbm, gathers_issued=2)

  @pl.when(jnp.logical_not(is_h))
  def _():
    rules_desc(0).wait()
    gather_desc(0, w_spmem).start()
    run_worker(NS_CHUNK, w_spmem, gathers_issued=1)


@jax.jit
def _run(rules_flat, relation, w_flat, bias_flat):
  mesh = plsc.VectorSubcoreMesh(
      core_axis_name="c", subcore_axis_name="s",
      num_cores=NC, num_subcores=NS)
  f = pl.kernel(
      _body,
      out_type=jax.ShapeDtypeStruct((B,), jnp.float32),
      mesh=mesh,
      compiler_params=pltpu.CompilerParams(needs_layout_passes=False),
      scratch_types=[
          pltpu.VMEM((CW,), jnp.int32),
          pltpu.VMEM((CW,), jnp.int32),
          pltpu.VMEM((CW,), jnp.float32),
          pltpu.VMEM((CW,), jnp.float32),
          pltpu.VMEM((SCW,), jnp.float32),
          pltpu.VMEM((SCW,), jnp.float32),
          pltpu.VMEM((NUM_REL,), jnp.float32),
          pltpu.VMEM((S_ROWS,), jnp.int32),
          pltpu.VMEM((S_ROWS,), jnp.float32),
          pltpu.VMEM_SHARED((NUM_W_PAD,), jnp.float32),
          pltpu.SemaphoreType.DMA,
          pltpu.SemaphoreType.DMA,
          pltpu.SemaphoreType.DMA,
          pltpu.SemaphoreType.DMA,
          pltpu.SemaphoreType.DMA,
          pltpu.SemaphoreType.DMA,
      ],
  )
  return f(rules_flat, relation, w_flat, bias_flat)


def kernel(rules, relation, rules_weight, bias):
  rules_flat = rules.astype(jnp.int32).reshape(B * L)
  relation = relation.astype(jnp.int32)
  w_flat = jnp.concatenate([
      rules_weight.reshape(NUM_W),
      jnp.zeros((NUM_W_PAD - NUM_W,), jnp.float32)])
  bias_flat = bias.reshape(NUM_REL)
  out = _run(rules_flat, relation, w_flat, bias_flat)
  return out.reshape(B, 1)


# speedup vs baseline: 1.1536x; 1.1536x over previous
import jax
import jax.numpy as jnp
from jax import lax
from jax.experimental import pallas as pl
from jax.experimental.pallas import tpu as pltpu
from jax.experimental.pallas import tpu_sc as plsc

B = 16384
L = 200
NUM_W = 1000001
NUM_REL = 1000

NC, NS, LANES = 2, 16, 16
NW = NC * NS
ROWS_PER_W = B // NW
CHUNK = 64
NCHUNK = ROWS_PER_W // CHUNK
CW = CHUNK * L
NGROUP = CHUNK // LANES

W_SLICE = 62504
NUM_W_PAD = W_SLICE * NS
SCW = 6400
N_STAGE = -(-W_SLICE // SCW)
STAGE_TAIL = W_SLICE - (N_STAGE - 1) * SCW


def _body(rules_hbm, rel_hbm, w_hbm, bias_hbm, out_hbm,
          rules_a, rules_b, vals_a, vals_b,
          bounce_a, bounce_b, bias_v, rel_v, out_acc,
          w_spmem, rsem_a, rsem_b, gsem_a, gsem_b, hsem, ssem):
  sid = lax.axis_index("s")
  wid = sid * NC + lax.axis_index("c")
  wbase = wid * ROWS_PER_W

  row_stride = lax.iota(jnp.int32, LANES) * L
  base_idx = [row_stride + g * (LANES * L) for g in range(NGROUP)]
  zero = jnp.zeros((LANES,), jnp.float32)

  rules_bufs = [rules_a, rules_b]
  vals_bufs = [vals_a, vals_b]
  rsem = [rsem_a, rsem_b]
  gsem = [gsem_a, gsem_b]
  bounce = [bounce_a, bounce_b]
  stage_n = [SCW] * (N_STAGE - 1) + [STAGE_TAIL]

  r_h, g_h, h_h = {}, {}, {}

  def issue_rules(c):
    p = c % 2
    r_h[c] = pltpu.async_copy(
        rules_hbm.at[pl.ds((wbase + c * CHUNK) * L, CW)], rules_bufs[p],
        rsem[p])

  def issue_gather(c):
    p = c % 2
    g_h[c] = pltpu.async_copy(w_spmem.at[rules_bufs[p]], vals_bufs[p],
                              gsem[p])

  def issue_stage_read(k):
    h_h[k] = pltpu.async_copy(
        w_hbm.at[pl.ds(sid * W_SLICE + k * SCW, stage_n[k])],
        bounce[k % 2].at[pl.ds(0, stage_n[k])], hsem)

  issue_rules(0)
  issue_rules(1)
  issue_stage_read(0)
  issue_stage_read(1)
  pltpu.sync_copy(bias_hbm, bias_v)
  pltpu.sync_copy(rel_hbm.at[pl.ds(wbase, ROWS_PER_W)], rel_v)

  for k in range(N_STAGE):
    h_h[k].wait()
    s = pltpu.async_copy(
        bounce[k % 2].at[pl.ds(0, stage_n[k])],
        w_spmem.at[pl.ds(sid * W_SLICE + k * SCW, stage_n[k])], ssem)
    s.wait()
    if k + 2 < N_STAGE:
      issue_stage_read(k + 2)

  plsc.subcore_barrier()

  r_h[0].wait()
  issue_gather(0)

  for c in range(NCHUNK):
    p = c % 2
    if c + 1 < NCHUNK:
      r_h[c + 1].wait()
      issue_gather(c + 1)
    g_h[c].wait()
    if c + 2 < NCHUNK:
      issue_rules(c + 2)

    vals_ref = vals_bufs[p]

    def l_body(l, accs, vals_ref=vals_ref):
      return tuple(
          accs[g] + plsc.load_gather(vals_ref, [base_idx[g] + l])
          for g in range(NGROUP))

    accs = lax.fori_loop(0, L, l_body, (zero,) * NGROUP, unroll=8)
    for g in range(NGROUP):
      out_acc[pl.ds(c * CHUNK + g * LANES, LANES)] = accs[g]

  def bias_body(g, carry):
    idx = rel_v[pl.ds(g * LANES, LANES)]
    out_acc[pl.ds(g * LANES, LANES)] = (
        out_acc[pl.ds(g * LANES, LANES)] + plsc.load_gather(bias_v, [idx]))
    return carry

  lax.fori_loop(0, ROWS_PER_W // LANES, bias_body, 0)

  pltpu.sync_copy(out_acc, out_hbm.at[pl.ds(wbase, ROWS_PER_W)])


@jax.jit
def _run(rules_flat, relation, w_flat, bias_flat):
  mesh = plsc.VectorSubcoreMesh(
      core_axis_name="c", subcore_axis_name="s",
      num_cores=NC, num_subcores=NS)
  f = pl.kernel(
      _body,
      out_type=jax.ShapeDtypeStruct((B,), jnp.float32),
      mesh=mesh,
      compiler_params=pltpu.CompilerParams(needs_layout_passes=False),
      scratch_types=[
          pltpu.VMEM((CW,), jnp.int32),
          pltpu.VMEM((CW,), jnp.int32),
          pltpu.VMEM((CW,), jnp.float32),
          pltpu.VMEM((CW,), jnp.float32),
          pltpu.VMEM((SCW,), jnp.float32),
          pltpu.VMEM((SCW,), jnp.float32),
          pltpu.VMEM((NUM_REL,), jnp.float32),
          pltpu.VMEM((ROWS_PER_W,), jnp.int32),
          pltpu.VMEM((ROWS_PER_W,), jnp.float32),
          pltpu.VMEM_SHARED((NUM_W_PAD,), jnp.float32),
          pltpu.SemaphoreType.DMA,
          pltpu.SemaphoreType.DMA,
          pltpu.SemaphoreType.DMA,
          pltpu.SemaphoreType.DMA,
          pltpu.SemaphoreType.DMA,
          pltpu.SemaphoreType.DMA,
      ],
  )
  return f(rules_flat, relation, w_flat, bias_flat)


def kernel(rules, relation, rules_weight, bias):
  rules_flat = rules.astype(jnp.int32).reshape(B * L)
  relation = relation.astype(jnp.int32)
  w_flat = jnp.concatenate([
      rules_weight.reshape(NUM_W),
      jnp.zeros((NUM_W_PAD - NUM_W,), jnp.float32)])
  bias_flat = bias.reshape(NUM_REL)
  out = _run(rules_flat, relation, w_flat, bias_flat)
  return out.reshape(B, 1)


# speedup vs baseline: 1.1539x; 1.0002x over previous
import jax
import jax.numpy as jnp
from jax import lax
from jax.experimental import pallas as pl
from jax.experimental.pallas import tpu as pltpu
from jax.experimental.pallas import tpu_sc as plsc

B = 16384
L = 200
NUM_W = 1000001
NUM_REL = 1000

NC, NS, LANES = 2, 16, 16
NW = NC * NS
ROWS_PER_W = B // NW
CHUNK = 64
NCHUNK = ROWS_PER_W // CHUNK
CW = CHUNK * L
NGROUP = CHUNK // LANES

W_SLICE = 62504
NUM_W_PAD = W_SLICE * NS
SCW = 6400
N_STAGE = -(-W_SLICE // SCW)
STAGE_TAIL = W_SLICE - (N_STAGE - 1) * SCW


def _body(rules_hbm, rel_hbm, w_hbm, bias_hbm, out_hbm,
          rules_a, rules_b, vals_a, vals_b,
          bounce_a, bounce_b, bias_v, rel_v, out_acc,
          w_spmem, rsem_a, rsem_b, gsem_a, gsem_b, hsem_a, hsem_b, ssem):
  sid = lax.axis_index("s")
  wid = sid * NC + lax.axis_index("c")
  wbase = wid * ROWS_PER_W

  row_stride = lax.iota(jnp.int32, LANES) * L
  base_idx = [row_stride + g * (LANES * L) for g in range(NGROUP)]
  zero = jnp.zeros((LANES,), jnp.float32)

  rules_bufs = [rules_a, rules_b]
  vals_bufs = [vals_a, vals_b]
  rsem = [rsem_a, rsem_b]
  gsem = [gsem_a, gsem_b]
  bounce = [bounce_a, bounce_b]
  hsem = [hsem_a, hsem_b]
  stage_n = [SCW] * (N_STAGE - 1) + [STAGE_TAIL]

  r_h, g_h, h_h = {}, {}, {}

  def issue_rules(c):
    p = c % 2
    r_h[c] = pltpu.async_copy(
        rules_hbm.at[pl.ds((wbase + c * CHUNK) * L, CW)], rules_bufs[p],
        rsem[p])

  def issue_gather(c):
    p = c % 2
    g_h[c] = pltpu.async_copy(w_spmem.at[rules_bufs[p]], vals_bufs[p],
                              gsem[p])

  def issue_stage_read(k):
    h_h[k] = pltpu.async_copy(
        w_hbm.at[pl.ds(sid * W_SLICE + k * SCW, stage_n[k])],
        bounce[k % 2].at[pl.ds(0, stage_n[k])], hsem[k % 2])

  issue_rules(0)
  issue_rules(1)
  issue_stage_read(0)
  issue_stage_read(1)
  pltpu.sync_copy(bias_hbm, bias_v)
  pltpu.sync_copy(rel_hbm.at[pl.ds(wbase, ROWS_PER_W)], rel_v)

  for k in range(N_STAGE):
    h_h[k].wait()
    s = pltpu.async_copy(
        bounce[k % 2].at[pl.ds(0, stage_n[k])],
        w_spmem.at[pl.ds(sid * W_SLICE + k * SCW, stage_n[k])], ssem)
    s.wait()
    if k + 2 < N_STAGE:
      issue_stage_read(k + 2)

  plsc.subcore_barrier()

  r_h[0].wait()
  issue_gather(0)

  for c in range(NCHUNK):
    p = c % 2
    if c + 1 < NCHUNK:
      r_h[c + 1].wait()
      issue_gather(c + 1)
    g_h[c].wait()
    if c + 2 < NCHUNK:
      issue_rules(c + 2)

    vals_ref = vals_bufs[p]

    def l_body(l, accs, vals_ref=vals_ref):
      return tuple(
          accs[g] + plsc.load_gather(vals_ref, [base_idx[g] + l])
          for g in range(NGROUP))

    accs = lax.fori_loop(0, L, l_body, (zero,) * NGROUP, unroll=8)
    for g in range(NGROUP):
      out_acc[pl.ds(c * CHUNK + g * LANES, LANES)] = accs[g]

  def bias_body(g, carry):
    idx = rel_v[pl.ds(g * LANES, LANES)]
    out_acc[pl.ds(g * LANES, LANES)] = (
        out_acc[pl.ds(g * LANES, LANES)] + plsc.load_gather(bias_v, [idx]))
    return carry

  lax.fori_loop(0, ROWS_PER_W // LANES, bias_body, 0)

  pltpu.sync_copy(out_acc, out_hbm.at[pl.ds(wbase, ROWS_PER_W)])


@jax.jit
def _run(rules_flat, relation, w_flat, bias_flat):
  mesh = plsc.VectorSubcoreMesh(
      core_axis_name="c", subcore_axis_name="s",
      num_cores=NC, num_subcores=NS)
  f = pl.kernel(
      _body,
      out_type=jax.ShapeDtypeStruct((B,), jnp.float32),
      mesh=mesh,
      compiler_params=pltpu.CompilerParams(needs_layout_passes=False),
      scratch_types=[
          pltpu.VMEM((CW,), jnp.int32),
          pltpu.VMEM((CW,), jnp.int32),
          pltpu.VMEM((CW,), jnp.float32),
          pltpu.VMEM((CW,), jnp.float32),
          pltpu.VMEM((SCW,), jnp.float32),
          pltpu.VMEM((SCW,), jnp.float32),
          pltpu.VMEM((NUM_REL,), jnp.float32),
          pltpu.VMEM((ROWS_PER_W,), jnp.int32),
          pltpu.VMEM((ROWS_PER_W,), jnp.float32),
          pltpu.VMEM_SHARED((NUM_W_PAD,), jnp.float32),
          pltpu.SemaphoreType.DMA,
          pltpu.SemaphoreType.DMA,
          pltpu.SemaphoreType.DMA,
          pltpu.SemaphoreType.DMA,
          pltpu.SemaphoreType.DMA,
          pltpu.SemaphoreType.DMA,
          pltpu.SemaphoreType.DMA,
      ],
  )
  return f(rules_flat, relation, w_flat, bias_flat)


def kernel(rules, relation, rules_weight, bias):
  rules_flat = rules.astype(jnp.int32).reshape(B * L)
  relation = relation.astype(jnp.int32)
  w_flat = jnp.concatenate([
      rules_weight.reshape(NUM_W),
      jnp.zeros((NUM_W_PAD - NUM_W,), jnp.float32)])
  bias_flat = bias.reshape(NUM_REL)
  out = _run(rules_flat, relation, w_flat, bias_flat)
  return out.reshape(B, 1)


# speedup vs baseline: 1.1572x; 1.0028x over previous
import jax
import jax.numpy as jnp
from jax import lax
from jax.experimental import pallas as pl
from jax.experimental.pallas import tpu as pltpu
from jax.experimental.pallas import tpu_sc as plsc

B = 16384
L = 200
NUM_W = 1000001
NUM_REL = 1000

NC, NS, LANES = 2, 16, 16
NW = NC * NS
ROWS_PER_W = B // NW
CHUNK = 64
NCHUNK = ROWS_PER_W // CHUNK
CW = CHUNK * L
NGROUP = CHUNK // LANES

W_SLICE = 62504
NUM_W_PAD = W_SLICE * NS
SCW = 6400
N_STAGE = -(-W_SLICE // SCW)
STAGE_TAIL = W_SLICE - (N_STAGE - 1) * SCW


def _body(rules_hbm, rel_hbm, w_hbm, bias_hbm, out_hbm,
          rules_a, rules_b, vals_a, vals_b,
          bounce_a, bounce_b, bias_v, rel_v, out_acc,
          w_spmem, rsem_a, rsem_b, gsem_a, gsem_b, hsem_a, hsem_b, ssem):
  sid = lax.axis_index("s")
  wid = sid * NC + lax.axis_index("c")
  wbase = wid * ROWS_PER_W

  row_stride = lax.iota(jnp.int32, LANES) * L
  base_idx = [row_stride + g * (LANES * L) for g in range(NGROUP)]
  zero = jnp.zeros((LANES,), jnp.float32)

  rules_bufs = [rules_a, rules_b]
  vals_bufs = [vals_a, vals_b]
  rsem = [rsem_a, rsem_b]
  gsem = [gsem_a, gsem_b]
  bounce = [bounce_a, bounce_b]
  hsem = [hsem_a, hsem_b]
  stage_n = [SCW] * (N_STAGE - 1) + [STAGE_TAIL]

  r_h, g_h, h_h = {}, {}, {}

  def issue_rules(c):
    p = c % 2
    r_h[c] = pltpu.async_copy(
        rules_hbm.at[pl.ds((wbase + c * CHUNK) * L, CW)], rules_bufs[p],
        rsem[p])

  def issue_gather(c):
    p = c % 2
    g_h[c] = pltpu.async_copy(w_spmem.at[rules_bufs[p]], vals_bufs[p],
                              gsem[p])

  def issue_stage_read(k):
    h_h[k] = pltpu.async_copy(
        w_hbm.at[pl.ds(sid * W_SLICE + k * SCW, stage_n[k])],
        bounce[k % 2].at[pl.ds(0, stage_n[k])], hsem[k % 2])

  issue_rules(0)
  issue_rules(1)
  issue_stage_read(0)
  issue_stage_read(1)
  pltpu.sync_copy(bias_hbm, bias_v)
  pltpu.sync_copy(rel_hbm.at[pl.ds(wbase, ROWS_PER_W)], rel_v)

  for k in range(N_STAGE):
    h_h[k].wait()
    s = pltpu.async_copy(
        bounce[k % 2].at[pl.ds(0, stage_n[k])],
        w_spmem.at[pl.ds(sid * W_SLICE + k * SCW, stage_n[k])], ssem)
    s.wait()
    if k + 2 < N_STAGE:
      issue_stage_read(k + 2)

  plsc.subcore_barrier()
  plsc.subcore_barrier()

  r_h[0].wait()
  issue_gather(0)

  for c in range(NCHUNK):
    p = c % 2
    g_h[c].wait()
    if c + 1 < NCHUNK:
      r_h[c + 1].wait()
      issue_gather(c + 1)
    if c + 2 < NCHUNK:
      issue_rules(c + 2)

    vals_ref = vals_bufs[p]

    def l_body(l, accs, vals_ref=vals_ref):
      return tuple(
          accs[g] + plsc.load_gather(vals_ref, [base_idx[g] + l])
          for g in range(NGROUP))

    accs = lax.fori_loop(0, L, l_body, (zero,) * NGROUP, unroll=8)
    for g in range(NGROUP):
      out_acc[pl.ds(c * CHUNK + g * LANES, LANES)] = accs[g]

  def bias_body(g, carry):
    idx = rel_v[pl.ds(g * LANES, LANES)]
    out_acc[pl.ds(g * LANES, LANES)] = (
        out_acc[pl.ds(g * LANES, LANES)] + plsc.load_gather(bias_v, [idx]))
    return carry

  lax.fori_loop(0, ROWS_PER_W // LANES, bias_body, 0)

  pltpu.sync_copy(out_acc, out_hbm.at[pl.ds(wbase, ROWS_PER_W)])


@jax.jit
def _run(rules_flat, relation, w_flat, bias_flat):
  mesh = plsc.VectorSubcoreMesh(
      core_axis_name="c", subcore_axis_name="s",
      num_cores=NC, num_subcores=NS)
  f = pl.kernel(
      _body,
      out_type=jax.ShapeDtypeStruct((B,), jnp.float32),
      mesh=mesh,
      compiler_params=pltpu.CompilerParams(needs_layout_passes=False),
      scratch_types=[
          pltpu.VMEM((CW,), jnp.int32),
          pltpu.VMEM((CW,), jnp.int32),
          pltpu.VMEM((CW,), jnp.float32),
          pltpu.VMEM((CW,), jnp.float32),
          pltpu.VMEM((SCW,), jnp.float32),
          pltpu.VMEM((SCW,), jnp.float32),
          pltpu.VMEM((NUM_REL,), jnp.float32),
          pltpu.VMEM((ROWS_PER_W,), jnp.int32),
          pltpu.VMEM((ROWS_PER_W,), jnp.float32),
          pltpu.VMEM_SHARED((NUM_W_PAD,), jnp.float32),
          pltpu.SemaphoreType.DMA,
          pltpu.SemaphoreType.DMA,
          pltpu.SemaphoreType.DMA,
          pltpu.SemaphoreType.DMA,
          pltpu.SemaphoreType.DMA,
          pltpu.SemaphoreType.DMA,
          pltpu.SemaphoreType.DMA,
      ],
  )
  return f(rules_flat, relation, w_flat, bias_flat)


def kernel(rules, relation, rules_weight, bias):
  rules_flat = rules.astype(jnp.int32).reshape(B * L)
  relation = relation.astype(jnp.int32)
  w_flat = jnp.concatenate([
      rules_weight.reshape(NUM_W),
      jnp.zeros((NUM_W_PAD - NUM_W,), jnp.float32)])
  bias_flat = bias.reshape(NUM_REL)
  out = _run(rules_flat, relation, w_flat, bias_flat)
  return out.reshape(B, 1)
